# baseline (speedup 1.0000x reference)
import functools

import jax
import jax.numpy as jnp
from jax import lax
from jax.experimental import pallas as pl
from jax.experimental.pallas import tpu as pltpu
from jax.experimental.pallas import tpu_sc as plsc

B, W, H, C = 8, 224, 224, 96
TASK = H * C
NTASK = B * C
NC, NS, L = 2, 16, 16
NW = NC * NS
TPW = NTASK // NW
NCHUNK = 96
VPC = TASK // (NCHUNK * L)
NEG_INF = float("-inf")
BIG = 2 ** 30


def _task_argmax(buf, cm):
    def chunk_body(c, carry):
        acc = jnp.full((L,), NEG_INF, jnp.float32)
        base = pl.multiple_of(c * (VPC * L), 8)
        for k in range(VPC):
            acc = jnp.maximum(acc, buf[pl.ds(base + k * L, L)])
        cm[pl.ds(pl.multiple_of(c * L, 8), L)] = acc
        return carry
    lax.fori_loop(0, NCHUNK, chunk_body, jnp.int32(0))

    def gm_body(i, acc):
        return jnp.maximum(acc, cm[pl.ds(pl.multiple_of(i * L, 8), L)])
    gacc = lax.fori_loop(0, NCHUNK, gm_body,
                         jnp.full((L,), NEG_INF, jnp.float32))
    m = jnp.max(gacc)

    def fc_body(i, acc):
        v = cm[pl.ds(pl.multiple_of(i * L, 8), L)]
        cand = jnp.where(v == m, jnp.broadcast_to(i, (L,)).astype(jnp.int32),
                         BIG)
        return jnp.minimum(acc, cand)
    fcand = lax.fori_loop(0, NCHUNK, fc_body, jnp.full((L,), BIG, jnp.int32))
    cstar = jnp.min(fcand)

    base = cstar * (VPC * L)
    lane = lax.iota(jnp.int32, L)
    def rs_body(k, acc):
        off = base + k * L
        v = buf[pl.ds(pl.multiple_of(off, 8), L)]
        cand = jnp.where(v == m, off + lane, BIG)
        return jnp.minimum(acc, cand)
    rcand = lax.fori_loop(0, VPC, rs_body, jnp.full((L,), BIG, jnp.int32))
    idx = jnp.min(rcand)

    col = (idx % W).astype(jnp.float32)
    row = (idx // W).astype(jnp.float32)
    return col, row


_mesh = plsc.VectorSubcoreMesh(core_axis_name="c", subcore_axis_name="s")


@functools.partial(
    pl.kernel,
    mesh=_mesh,
    out_type=jax.ShapeDtypeStruct((NTASK * 2,), jnp.float32),
    scratch_types=[
        pltpu.VMEM((TASK,), jnp.float32),
        pltpu.VMEM((TASK,), jnp.float32),
        pltpu.VMEM((NCHUNK * L,), jnp.float32),
        pltpu.VMEM((2 * TPW,), jnp.float32),
        pltpu.SemaphoreType.DMA,
        pltpu.SemaphoreType.DMA,
    ],
    compiler_params=pltpu.CompilerParams(needs_layout_passes=False),
)
def _sc_argmax(x_hbm, out_hbm, buf0, buf1, cm, outv, sem0, sem1):
    wid = lax.axis_index("c") * NS + lax.axis_index("s")
    t0 = wid * TPW
    bufs = (buf0, buf1)
    sems = (sem0, sem1)

    def start(t):
        tg = t0 + t
        b = tg // C
        j = tg % C
        off = pl.multiple_of((b * W + j) * TASK, 8)
        return pltpu.async_copy(x_hbm.at[pl.ds(off, TASK)],
                                bufs[t % 2], sems[t % 2])

    lane = lax.iota(jnp.int32, L)
    accs = [jnp.zeros((L,), jnp.float32) for _ in range(2 * TPW // L)]
    handles = [start(0), None]
    for t in range(TPW):
        handles[t % 2].wait()
        if t + 1 < TPW:
            handles[(t + 1) % 2] = start(t + 1)
        col, row = _task_argmax(bufs[t % 2], cm)
        vi, p = (2 * t) // L, (2 * t) % L
        acc = jnp.where(lane == p, col, accs[vi])
        accs[vi] = jnp.where(lane == p + 1, row, acc)

    for i, acc in enumerate(accs):
        outv[pl.ds(i * L, L)] = acc
    out_off = pl.multiple_of(t0 * 2, 8)
    pltpu.sync_copy(outv, out_hbm.at[pl.ds(out_off, 2 * TPW)])


def kernel(x):
    out_flat = _sc_argmax(x.reshape(-1))
    return out_flat.reshape(B, 2 * C)


# speedup vs baseline: 4.1544x; 4.1544x over previous
import functools

import jax
import jax.numpy as jnp
from jax import lax
from jax.experimental import pallas as pl
from jax.experimental.pallas import tpu as pltpu
from jax.experimental.pallas import tpu_sc as plsc

B, W, H, C = 8, 224, 224, 96
TASK = H * C
NTASK = B * C
NC, NS, L = 2, 16, 16
NW = NC * NS
TPW = NTASK // NW
VPR = C // L
RPC = 4
NCHUNK = H // RPC
NEG_INF = float("-inf")
BIG = 2 ** 30


def _task_argmax(buf, cm):
    def chunk_body(c, gacc):
        r0 = c * RPC
        accs = [buf[r0, pl.ds(k * L, L)] for k in range(4)]
        for u in range(RPC):
            for k in range(VPR):
                if u == 0 and k < 4:
                    continue
                accs[k % 4] = jnp.maximum(accs[k % 4],
                                          buf[r0 + u, pl.ds(k * L, L)])
        acc = jnp.maximum(jnp.maximum(accs[0], accs[1]),
                          jnp.maximum(accs[2], accs[3]))
        cm[pl.ds(pl.multiple_of(c * L, 8), L)] = acc
        return jnp.maximum(gacc, acc)
    gacc = lax.fori_loop(0, NCHUNK, chunk_body,
                         jnp.full((L,), NEG_INF, jnp.float32))
    m = jnp.max(gacc)

    big = jnp.full((L,), BIG, jnp.int32)
    def fc_body(i, carry):
        a0, a1 = carry
        c0 = i * 8
        for u in range(8):
            v = cm[pl.ds(pl.multiple_of((c0 + u) * L, 8), L)]
            cand = jnp.where(v == m, c0 + u, BIG)
            if u % 2 == 0:
                a0 = jnp.minimum(a0, cand)
            else:
                a1 = jnp.minimum(a1, cand)
        return a0, a1
    a0, a1 = lax.fori_loop(0, NCHUNK // 8, fc_body, (big, big))
    cstar = jnp.min(jnp.minimum(a0, a1))

    lane = lax.iota(jnp.int32, L)
    def rs_body(u, carry):
        ra, rb = carry
        r = cstar * RPC + u
        for k in range(VPR):
            v = buf[r, pl.ds(k * L, L)]
            cand = jnp.where(v == m, r * C + k * L + lane, BIG)
            if k % 2 == 0:
                ra = jnp.minimum(ra, cand)
            else:
                rb = jnp.minimum(rb, cand)
        return ra, rb
    ra, rb = lax.fori_loop(0, RPC, rs_body, (big, big))
    idx = jnp.min(jnp.minimum(ra, rb))

    col = (idx % W).astype(jnp.float32)
    row = (idx // W).astype(jnp.float32)
    return col, row


_mesh = plsc.VectorSubcoreMesh(core_axis_name="c", subcore_axis_name="s")


@functools.partial(
    pl.kernel,
    mesh=_mesh,
    out_type=jax.ShapeDtypeStruct((NTASK * 2,), jnp.float32),
    scratch_types=[
        pltpu.VMEM((H, C), jnp.float32),
        pltpu.VMEM((H, C), jnp.float32),
        pltpu.VMEM((NCHUNK * L,), jnp.float32),
        pltpu.VMEM((2 * TPW,), jnp.float32),
        pltpu.SemaphoreType.DMA,
        pltpu.SemaphoreType.DMA,
    ],
    compiler_params=pltpu.CompilerParams(
        needs_layout_passes=False, use_tc_tiling_on_sc=True),
)
def _sc_argmax(x_hbm, out_hbm, buf0, buf1, cm, outv, sem0, sem1):
    wid = lax.axis_index("c") * NS + lax.axis_index("s")
    t0 = wid * TPW
    lane = lax.iota(jnp.int32, L)

    def start(tg, buf, sem):
        pltpu.async_copy(x_hbm.at[tg // C, tg % C], buf, sem)

    def wait(buf, sem):
        pltpu.make_async_copy(x_hbm.at[0, 0], buf, sem).wait()

    start(t0, buf0, sem0)
    start(t0 + 1, buf1, sem1)

    def pair_body(q, carry):
        for s_ in range(2):
            buf = (buf0, buf1)[s_]
            sem = (sem0, sem1)[s_]
            t = 2 * q + s_
            wait(buf, sem)
            col, row = _task_argmax(buf, cm)

            @pl.when(q < TPW // 2 - 1)
            def _():
                start(t0 + t + 2, buf, sem)

            val = jnp.where(lane == 0, col, row)
            plsc.store_scatter(outv, [2 * t + lane], val, mask=lane < 2)
        return carry
    lax.fori_loop(0, TPW // 2, pair_body, jnp.int32(0))

    out_off = pl.multiple_of(t0 * 2, 8)
    pltpu.sync_copy(outv, out_hbm.at[pl.ds(out_off, 2 * TPW)])


def kernel(x):
    out_flat = _sc_argmax(x)
    return out_flat.reshape(B, 2 * C)


# speedup vs baseline: 15.4503x; 3.7191x over previous
import functools

import jax
import jax.numpy as jnp
from jax import lax
from jax.experimental import pallas as pl
from jax.experimental.pallas import tpu as pltpu
from jax.experimental.pallas import tpu_sc as plsc

B, W, H, C = 8, 224, 224, 96
TASK = H * C
NTASK = B * C
NC, NS, L = 2, 16, 16
NW = NC * NS
TPW = NTASK // NW
VPR = H // L
RPC = 2
NCHUNK = C // RPC
NEG_INF = float("-inf")
BIG = 2 ** 30


def _task_argmax(buf, cm):
    lane = lax.iota(jnp.int32, L)
    big = jnp.full((L,), BIG, jnp.int32)

    def chunk_body(c, gacc):
        r0 = c * RPC
        accs = [buf[r0, pl.ds(k * L, L)] for k in range(4)]
        for u in range(RPC):
            for k in range(VPR):
                if u == 0 and k < 4:
                    continue
                accs[k % 4] = jnp.maximum(accs[k % 4],
                                          buf[r0 + u, pl.ds(k * L, L)])
        acc = jnp.maximum(jnp.maximum(accs[0], accs[1]),
                          jnp.maximum(accs[2], accs[3]))
        cm[pl.ds(pl.multiple_of(c * L, 8), L)] = acc
        return jnp.maximum(gacc, acc)
    gacc = lax.fori_loop(0, NCHUNK, chunk_body,
                         jnp.full((L,), NEG_INF, jnp.float32))
    m = jnp.max(gacc)

    def fc_body(i, carry):
        alo, ahi = carry
        c0 = i * 8
        for u in range(8):
            v = cm[pl.ds(pl.multiple_of((c0 + u) * L, 8), L)]
            eq = v == m
            alo = jnp.minimum(alo, jnp.where(eq, c0 + u, BIG))
            ahi = jnp.maximum(ahi, jnp.where(eq, c0 + u, -1))
        return alo, ahi
    alo, ahi = lax.fori_loop(0, NCHUNK // 8, fc_body,
                             (big, jnp.full((L,), -1, jnp.int32)))
    cstar = jnp.min(alo)
    cmax = jnp.max(ahi)

    def scan_chunk(c, carry):
        ra, rb = carry
        for u in range(RPC):
            r = c * RPC + u
            for k in range(VPR):
                v = buf[r, pl.ds(k * L, L)]
                q = (k * L + lane) * C + r
                cand = jnp.where(v == m, q, BIG)
                if k % 2 == 0:
                    ra = jnp.minimum(ra, cand)
                else:
                    rb = jnp.minimum(rb, cand)
        return ra, rb

    def one_chunk():
        ra, rb = scan_chunk(cstar, (big, big))
        return jnp.min(jnp.minimum(ra, rb))

    def all_chunks():
        ra, rb = lax.fori_loop(0, NCHUNK, scan_chunk, (big, big))
        return jnp.min(jnp.minimum(ra, rb))

    idx = lax.cond(cmax == cstar, one_chunk, all_chunks)
    col = (idx % W).astype(jnp.float32)
    row = (idx // W).astype(jnp.float32)
    return col, row


_mesh = plsc.VectorSubcoreMesh(core_axis_name="c", subcore_axis_name="s")


@functools.partial(
    pl.kernel,
    mesh=_mesh,
    out_type=jax.ShapeDtypeStruct((NTASK * 2,), jnp.float32),
    scratch_types=[
        pltpu.VMEM((C, H), jnp.float32),
        pltpu.VMEM((C, H), jnp.float32),
        pltpu.VMEM((NCHUNK * L,), jnp.float32),
        pltpu.VMEM((2 * TPW,), jnp.float32),
        pltpu.SemaphoreType.DMA,
        pltpu.SemaphoreType.DMA,
    ],
    compiler_params=pltpu.CompilerParams(
        needs_layout_passes=False, use_tc_tiling_on_sc=True),
)
def _sc_argmax(xt_hbm, out_hbm, buf0, buf1, cm, outv, sem0, sem1):
    wid = lax.axis_index("c") * NS + lax.axis_index("s")
    t0 = wid * TPW
    lane = lax.iota(jnp.int32, L)

    def start(tg, buf, sem):
        pltpu.async_copy(xt_hbm.at[tg // C, tg % C], buf, sem)

    def wait(buf, sem):
        pltpu.make_async_copy(xt_hbm.at[0, 0], buf, sem).wait()

    start(t0, buf0, sem0)
    start(t0 + 1, buf1, sem1)

    def pair_body(q, carry):
        for s_ in range(2):
            buf = (buf0, buf1)[s_]
            sem = (sem0, sem1)[s_]
            t = 2 * q + s_
            wait(buf, sem)
            col, row = _task_argmax(buf, cm)

            @pl.when(q < TPW // 2 - 1)
            def _():
                start(t0 + t + 2, buf, sem)

            val = jnp.where(lane == 0, col, row)
            plsc.store_scatter(outv, [2 * t + lane], val, mask=lane < 2)
        return carry
    lax.fori_loop(0, TPW // 2, pair_body, jnp.int32(0))

    out_off = pl.multiple_of(t0 * 2, 8)
    pltpu.sync_copy(outv, out_hbm.at[pl.ds(out_off, 2 * TPW)])


def kernel(x):
    out_flat = _sc_argmax(jnp.swapaxes(x, 2, 3))
    return out_flat.reshape(B, 2 * C)
